# Initial kernel scaffold; baseline (speedup 1.0000x reference)
import jax
import jax.numpy as jnp
from jax.experimental import pallas as pl


def kernel(x, e_i, e_a, Wl1, bl1, Wr1, br1, We1, att1, b1, Wl2, bl2, Wr2, br2, We2, att2, b2):
    raise NotImplementedError("write your pallas kernel here")


# speedup vs baseline: 7.7957x; 7.7957x over previous
import functools
import math

import jax
import jax.numpy as jnp
from jax import lax
from jax.experimental import pallas as pl
from jax.experimental.pallas import tpu as pltpu
from jax.experimental.pallas import tpu_sc as plsc

_N = 10000
_E = 320000
_D = 128
_NC = 2
_NS = 16
_NW = _NC * _NS
_L = 16
_NP = 10240
_SLC = _NP // _NS
_EPT = _E // _NW
_CH = 80
_NCHUNK = _EPT // _CH
_NGRP = _N // _L

_mesh = plsc.VectorSubcoreMesh(
    core_axis_name="c", subcore_axis_name="s", num_cores=_NC, num_subcores=_NS
)

_f32 = jnp.float32
_i32 = jnp.int32


def _zero_rows(buf, nrows):
    z = jnp.zeros((_L,), _f32)

    def body(i, _):
        for k in range(_D // _L):
            buf[i, pl.ds(k * _L, _L)] = z
        return 0

    lax.fori_loop(0, nrows, body, 0)


def _zero_flat(buf, n):
    z = jnp.zeros((_L,), _f32)

    def body(i, _):
        buf[pl.ds(i * _L, _L)] = z
        return 0

    lax.fori_loop(0, n // _L, body, 0)


@functools.partial(
    pl.kernel,
    out_type=(
        jax.ShapeDtypeStruct((_NC, _NP), _f32),
        jax.ShapeDtypeStruct((_NC, _NP), _f32),
    ),
    mesh=_mesh,
    scratch_types=dict(
        deg_sh=pltpu.VMEM_SHARED((_NP,), _f32),
        eas_sh=pltpu.VMEM_SHARED((_NP,), _f32),
        dstb=pltpu.VMEM((_CH,), _i32),
        eab=pltpu.VMEM((_CH,), _f32),
        onesb=pltpu.VMEM((_CH,), _f32),
        zb=pltpu.VMEM((_SLC,), _f32),
    ),
)
def _prologue(dst_hbm, ea_hbm, deg_out, eas_out, *, deg_sh, eas_sh, dstb, eab, onesb, zb):
    c = lax.axis_index("c")
    s = lax.axis_index("s")
    wid = c * _NS + s

    one = jnp.ones((_L,), _f32)
    for i in range(_CH // _L):
        onesb[pl.ds(i * _L, _L)] = one
    _zero_flat(zb, _SLC)
    row0 = s * _SLC
    pltpu.sync_copy(zb, deg_sh.at[pl.ds(row0, _SLC)])
    pltpu.sync_copy(zb, eas_sh.at[pl.ds(row0, _SLC)])
    plsc.subcore_barrier()

    ebase = wid * _EPT

    def chunk(ci, _):
        base = ebase + ci * _CH
        pltpu.sync_copy(dst_hbm.at[pl.ds(base, _CH)], dstb)
        pltpu.sync_copy(ea_hbm.at[pl.ds(base, _CH)], eab)
        pltpu.sync_copy(onesb, deg_sh.at[dstb], add=True)
        pltpu.sync_copy(eab, eas_sh.at[dstb], add=True)
        return 0

    lax.fori_loop(0, _NCHUNK, chunk, 0)
    plsc.subcore_barrier()
    pltpu.sync_copy(deg_sh.at[pl.ds(row0, _SLC)], deg_out.at[c, pl.ds(row0, _SLC)])
    pltpu.sync_copy(eas_sh.at[pl.ds(row0, _SLC)], eas_out.at[c, pl.ds(row0, _SLC)])


@functools.partial(
    pl.kernel,
    out_type=(
        jax.ShapeDtypeStruct((_NC, _NP, _D), _f32),
        jax.ShapeDtypeStruct((_NC, _NP), _f32),
    ),
    mesh=_mesh,
    scratch_types=dict(
        u_sh=pltpu.VMEM_SHARED((_NP, _D), _f32),
        as_sh=pltpu.VMEM_SHARED((_NP,), _f32),
        srcb=pltpu.VMEM((_CH,), _i32),
        dstb=pltpu.VMEM((_CH,), _i32),
        eab=pltpu.VMEM((_CH,), _f32),
        sb=pltpu.VMEM((_CH,), _f32),
        xlb=pltpu.VMEM((_CH, _D), _f32),
        xrb=pltpu.VMEM((_CH, _D), _f32),
        pb=pltpu.VMEM((_CH, _D), _f32),
        wb=pltpu.VMEM((_D,), _f32),
        ab=pltpu.VMEM((_D,), _f32),
        zrow=pltpu.VMEM((_L, _D), _f32),
        azb=pltpu.VMEM((_SLC,), _f32),
        lad=pltpu.VMEM((_L, 2 * _L), _f32),
        xls=pltpu.VMEM((_L, _D), _f32),
        xrs=pltpu.VMEM((_L, _D), _f32),
        prods=pltpu.VMEM((_L, _D), _f32),
        ssb=pltpu.VMEM((_L,), _f32),
        idxb=pltpu.VMEM((_L,), _i32),
        d0b=pltpu.VMEM((_L,), _f32),
        d1b=pltpu.VMEM((_L,), _f32),
        e0b=pltpu.VMEM((_L,), _f32),
        e1b=pltpu.VMEM((_L,), _f32),
        sem1=pltpu.SemaphoreType.DMA,
        sem2=pltpu.SemaphoreType.DMA,
    ),
)
def _layer_sc(
    xl_hbm, xr_hbm, src_hbm, dst_hbm, ea_hbm, deg_hbm, eas_hbm, w_hbm, a_hbm,
    u_out, as_out, *,
    u_sh, as_sh, srcb, dstb, eab, sb, xlb, xrb, pb, wb, ab, zrow, azb,
    lad, xls, xrs, prods, ssb, idxb, d0b, d1b, e0b, e1b, sem1, sem2,
):
    c = lax.axis_index("c")
    s = lax.axis_index("s")
    wid = c * _NS + s
    lane = lax.iota(_i32, _L)

    _zero_rows(zrow, _L)
    _zero_flat(azb, _SLC)
    row0 = s * _SLC

    def zcp(r, _):
        pltpu.sync_copy(zrow, u_sh.at[pl.ds(row0 + r * _L, _L), :])
        return 0

    lax.fori_loop(0, _SLC // _L, zcp, 0)
    pltpu.sync_copy(azb, as_sh.at[pl.ds(row0, _SLC)])
    plsc.subcore_barrier()

    pltpu.sync_copy(w_hbm, wb)
    pltpu.sync_copy(a_hbm, ab)
    wv = [wb[pl.ds(k * _L, _L)] for k in range(_D // _L)]
    av = [ab[pl.ds(k * _L, _L)] for k in range(_D // _L)]
    zv = jnp.zeros((_L,), _f32)
    for e in range(_L):
        lad[e, pl.ds(_L, _L)] = zv

    def _attention_groups(ngroups, xsrc, xdst, get_ea16, sbuf, pbuf):
        def group(g, _):
            ea16 = get_ea16(g)
            alpha = zv
            for e in range(_L):
                eidx = g * _L + e
                eav = jnp.full((_L,), ea16[e], _f32)
                acc = None
                for k in range(_D // _L):
                    m = xsrc[eidx, pl.ds(k * _L, _L)] + xdst[eidx, pl.ds(k * _L, _L)]
                    m = m + eav * wv[k]
                    m = jnp.maximum(m, 0.2 * m)
                    t = m * av[k]
                    acc = t if acc is None else acc + t
                red = acc
                for sh in (8, 4, 2, 1):
                    lad[e, pl.ds(0, _L)] = red
                    red = red + lad[e, pl.ds(sh, _L)]
                alpha = jnp.where(lane == e, jnp.full((_L,), red[0], _f32), alpha)
            sv = jnp.exp(alpha)
            sbuf[pl.ds(g * _L, _L)] = sv
            for e in range(_L):
                eidx = g * _L + e
                se = jnp.full((_L,), sv[e], _f32)
                for k in range(_D // _L):
                    pbuf[eidx, pl.ds(k * _L, _L)] = xsrc[eidx, pl.ds(k * _L, _L)] * se
            return 0

        lax.fori_loop(0, ngroups, group, 0)

    ebase = wid * _EPT

    def chunk(ci, _):
        base = ebase + ci * _CH
        pltpu.sync_copy(src_hbm.at[pl.ds(base, _CH)], srcb)
        pltpu.sync_copy(dst_hbm.at[pl.ds(base, _CH)], dstb)
        pltpu.sync_copy(ea_hbm.at[pl.ds(base, _CH)], eab)
        cp1 = pltpu.async_copy(xl_hbm.at[srcb], xlb, sem1)
        cp2 = pltpu.async_copy(xr_hbm.at[dstb], xrb, sem2)
        cp1.wait()
        cp2.wait()
        _attention_groups(
            _CH // _L, xlb, xrb, lambda g: eab[pl.ds(g * _L, _L)], sb, pb
        )
        pltpu.sync_copy(sb, as_sh.at[dstb], add=True)
        pltpu.sync_copy(pb, u_sh.at[dstb], add=True)
        return 0

    lax.fori_loop(0, _NCHUNK, chunk, 0)

    g0 = wid * _NGRP // _NW
    g1 = (wid + 1) * _NGRP // _NW

    def sgroup(g, _):
        nb = g * _L
        pltpu.sync_copy(deg_hbm.at[0, pl.ds(nb, _L)], d0b)
        pltpu.sync_copy(deg_hbm.at[1, pl.ds(nb, _L)], d1b)
        pltpu.sync_copy(eas_hbm.at[0, pl.ds(nb, _L)], e0b)
        pltpu.sync_copy(eas_hbm.at[1, pl.ds(nb, _L)], e1b)
        deg = d0b[...] + d1b[...]
        lea = (e0b[...] + e1b[...]) / jnp.maximum(deg, 1.0)
        idxb[...] = lane + nb
        pltpu.sync_copy(xl_hbm.at[pl.ds(nb, _L), :], xls)
        pltpu.sync_copy(xr_hbm.at[pl.ds(nb, _L), :], xrs)
        _attention_groups(1, xls, xrs, lambda g: lea, ssb, prods)
        pltpu.sync_copy(ssb, as_sh.at[idxb], add=True)
        pltpu.sync_copy(prods, u_sh.at[idxb], add=True)
        return 0

    lax.fori_loop(g0, g1, sgroup, 0)

    plsc.subcore_barrier()
    for r in range(_SLC // _D):
        pltpu.sync_copy(
            u_sh.at[pl.ds(row0 + r * _D, _D), :],
            u_out.at[c, pl.ds(row0 + r * _D, _D), :],
        )
    pltpu.sync_copy(as_sh.at[pl.ds(row0, _SLC)], as_out.at[c, pl.ds(row0, _SLC)])


_BLK = 640


def _proj_body(x_ref, wl_ref, bl_ref, wr_ref, br_ref, xl_ref, xr_ref):
    xv = x_ref[...]
    xl_ref[...] = jnp.dot(xv, wl_ref[...], preferred_element_type=_f32) + bl_ref[...]
    xr_ref[...] = jnp.dot(xv, wr_ref[...], preferred_element_type=_f32) + br_ref[...]


def _proj(x, wl, bl, wr, br):
    n = x.shape[0]
    blk = 1000
    assert n % blk == 0
    grid = n // blk
    wspec = pl.BlockSpec((_D, _D), lambda i: (0, 0))
    bspec = pl.BlockSpec((1, _D), lambda i: (0, 0))
    rspec = pl.BlockSpec((blk, _D), lambda i: (i, 0))
    return pl.pallas_call(
        _proj_body,
        grid=(grid,),
        in_specs=[rspec, wspec, bspec, wspec, bspec],
        out_specs=[rspec, rspec],
        out_shape=(
            jax.ShapeDtypeStruct((n, _D), _f32),
            jax.ShapeDtypeStruct((n, _D), _f32),
        ),
    )(x, wl, bl.reshape(1, _D), wr, br.reshape(1, _D))


def _fin_proj_body(u_ref, as_ref, b_ref, wl_ref, bl_ref, wr_ref, br_ref, xl_ref, xr_ref):
    usum = u_ref[0] + u_ref[1]
    ssum = as_ref[0] + as_ref[1]
    h = usum / jnp.maximum(ssum, 1e-35) + b_ref[...]
    g = 0.5 * h * (1.0 + lax.erf(h * (1.0 / math.sqrt(2.0))))
    xl_ref[...] = jnp.dot(g, wl_ref[...], preferred_element_type=_f32) + bl_ref[...]
    xr_ref[...] = jnp.dot(g, wr_ref[...], preferred_element_type=_f32) + br_ref[...]


def _fin_proj(u, asum, b, wl, bl, wr, br):
    grid = _NP // _BLK
    uspec = pl.BlockSpec((_NC, _BLK, _D), lambda i: (0, i, 0))
    aspec = pl.BlockSpec((_NC, _BLK, 1), lambda i: (0, i, 0))
    wspec = pl.BlockSpec((_D, _D), lambda i: (0, 0))
    bspec = pl.BlockSpec((1, _D), lambda i: (0, 0))
    rspec = pl.BlockSpec((_BLK, _D), lambda i: (i, 0))
    return pl.pallas_call(
        _fin_proj_body,
        grid=(grid,),
        in_specs=[uspec, aspec, bspec, wspec, bspec, wspec, bspec],
        out_specs=[rspec, rspec],
        out_shape=(
            jax.ShapeDtypeStruct((_NP, _D), _f32),
            jax.ShapeDtypeStruct((_NP, _D), _f32),
        ),
    )(u, asum.reshape(_NC, _NP, 1), b.reshape(1, _D), wl, bl.reshape(1, _D), wr, br.reshape(1, _D))


def _fin_body(u_ref, as_ref, b_ref, o_ref):
    usum = u_ref[0] + u_ref[1]
    ssum = as_ref[0] + as_ref[1]
    o_ref[...] = usum / jnp.maximum(ssum, 1e-35) + b_ref[...]


def _fin(u, asum, b):
    grid = _NP // _BLK
    uspec = pl.BlockSpec((_NC, _BLK, _D), lambda i: (0, i, 0))
    aspec = pl.BlockSpec((_NC, _BLK, 1), lambda i: (0, i, 0))
    bspec = pl.BlockSpec((1, _D), lambda i: (0, 0))
    rspec = pl.BlockSpec((_BLK, _D), lambda i: (i, 0))
    return pl.pallas_call(
        _fin_body,
        grid=(grid,),
        in_specs=[uspec, aspec, bspec],
        out_specs=rspec,
        out_shape=jax.ShapeDtypeStruct((_NP, _D), _f32),
    )(u, asum.reshape(_NC, _NP, 1), b.reshape(1, _D))


def kernel(x, e_i, e_a, Wl1, bl1, Wr1, br1, We1, att1, b1, Wl2, bl2, Wr2, br2, We2, att2, b2):
    src = e_i[0]
    dst = e_i[1]
    ea = e_a.reshape(-1)
    deg_p, eas_p = _prologue(dst, ea)
    xl1, xr1 = _proj(x, Wl1, bl1, Wr1, br1)
    u1, as1 = _layer_sc(
        xl1, xr1, src, dst, ea, deg_p, eas_p, We1.reshape(-1), att1.reshape(-1)
    )
    xl2, xr2 = _fin_proj(u1, as1, b1, Wl2, bl2, Wr2, br2)
    u2, as2 = _layer_sc(
        xl2, xr2, src, dst, ea, deg_p, eas_p, We2.reshape(-1), att2.reshape(-1)
    )
    out = _fin(u2, as2, b2)
    return out[:_N]


# speedup vs baseline: 10.7911x; 1.3842x over previous
import functools
import math

import jax
import jax.numpy as jnp
from jax import lax
from jax.experimental import pallas as pl
from jax.experimental.pallas import tpu as pltpu
from jax.experimental.pallas import tpu_sc as plsc

_N = 10000
_E = 320000
_D = 128
_NC = 2
_NS = 16
_NW = _NC * _NS
_L = 16
_NP = 10240
_SLC = _NP // _NS
_EPT = _E // _NW
_CH = 80
_NCHUNK = _EPT // _CH
_NGRP = _N // _L

_mesh = plsc.VectorSubcoreMesh(
    core_axis_name="c", subcore_axis_name="s", num_cores=_NC, num_subcores=_NS
)

_f32 = jnp.float32
_i32 = jnp.int32


def _zero_rows(buf, nrows):
    z = jnp.zeros((_L,), _f32)

    def body(i, _):
        for k in range(_D // _L):
            buf[i, pl.ds(k * _L, _L)] = z
        return 0

    lax.fori_loop(0, nrows, body, 0)


def _zero_flat(buf, n):
    z = jnp.zeros((_L,), _f32)

    def body(i, _):
        buf[pl.ds(i * _L, _L)] = z
        return 0

    lax.fori_loop(0, n // _L, body, 0)


@functools.partial(
    pl.kernel,
    out_type=(
        jax.ShapeDtypeStruct((_NC * _NP,), _f32),
        jax.ShapeDtypeStruct((_NC * _NP,), _f32),
    ),
    mesh=_mesh,
    scratch_types=dict(
        deg_sh=pltpu.VMEM_SHARED((_NP,), _f32),
        eas_sh=pltpu.VMEM_SHARED((_NP,), _f32),
        dstb=pltpu.VMEM((_CH,), _i32),
        eab=pltpu.VMEM((_CH,), _f32),
        onesb=pltpu.VMEM((_CH,), _f32),
        zb=pltpu.VMEM((_SLC,), _f32),
    ),
)
def _prologue(dst_hbm, ea_hbm, deg_out, eas_out, *, deg_sh, eas_sh, dstb, eab, onesb, zb):
    c = lax.axis_index("c")
    s = lax.axis_index("s")
    wid = c * _NS + s

    one = jnp.ones((_L,), _f32)
    for i in range(_CH // _L):
        onesb[pl.ds(i * _L, _L)] = one
    _zero_flat(zb, _SLC)
    row0 = s * _SLC
    pltpu.sync_copy(zb, deg_sh.at[pl.ds(row0, _SLC)])
    pltpu.sync_copy(zb, eas_sh.at[pl.ds(row0, _SLC)])
    plsc.subcore_barrier()

    ebase = wid * _EPT

    def chunk(ci, _):
        base = ebase + ci * _CH
        pltpu.sync_copy(dst_hbm.at[pl.ds(base, _CH)], dstb)
        pltpu.sync_copy(ea_hbm.at[pl.ds(base, _CH)], eab)
        pltpu.sync_copy(onesb, deg_sh.at[dstb], add=True)
        pltpu.sync_copy(eab, eas_sh.at[dstb], add=True)
        return 0

    lax.fori_loop(0, _NCHUNK, chunk, 0)
    plsc.subcore_barrier()
    pltpu.sync_copy(deg_sh.at[pl.ds(row0, _SLC)], deg_out.at[pl.ds(c * _NP + row0, _SLC)])
    pltpu.sync_copy(eas_sh.at[pl.ds(row0, _SLC)], eas_out.at[pl.ds(c * _NP + row0, _SLC)])


_NSC = _N // _CH


@functools.partial(
    pl.kernel,
    out_type=(
        jax.ShapeDtypeStruct((_NC, _NP, _D), _f32),
        jax.ShapeDtypeStruct((_NC * _NP,), _f32),
    ),
    mesh=_mesh,
    scratch_types=dict(
        u_sh=pltpu.VMEM_SHARED((_NP, _D), _f32),
        as_sh=pltpu.VMEM_SHARED((_NP,), _f32),
        xlb0=pltpu.VMEM((_CH, _D), _f32),
        xlb1=pltpu.VMEM((_CH, _D), _f32),
        xrb0=pltpu.VMEM((_CH, _D), _f32),
        xrb1=pltpu.VMEM((_CH, _D), _f32),
        srcb0=pltpu.VMEM((_CH,), _i32),
        srcb1=pltpu.VMEM((_CH,), _i32),
        dstb0=pltpu.VMEM((_CH,), _i32),
        dstb1=pltpu.VMEM((_CH,), _i32),
        eab0=pltpu.VMEM((_CH,), _f32),
        eab1=pltpu.VMEM((_CH,), _f32),
        sb0=pltpu.VMEM((_CH,), _f32),
        sb1=pltpu.VMEM((_CH,), _f32),
        wb=pltpu.VMEM((_D,), _f32),
        ab=pltpu.VMEM((_D,), _f32),
        azb=pltpu.VMEM((_SLC,), _f32),
        lad=pltpu.VMEM((_L, 2 * _L), _f32),
        idxb=pltpu.VMEM((_CH,), _i32),
        d0b=pltpu.VMEM((_CH,), _f32),
        d1b=pltpu.VMEM((_CH,), _f32),
        e0b=pltpu.VMEM((_CH,), _f32),
        e1b=pltpu.VMEM((_CH,), _f32),
        mm0=pltpu.SemaphoreType.DMA,
        mm1=pltpu.SemaphoreType.DMA,
        gx0=pltpu.SemaphoreType.DMA,
        gx1=pltpu.SemaphoreType.DMA,
        gr0=pltpu.SemaphoreType.DMA,
        gr1=pltpu.SemaphoreType.DMA,
        us0=pltpu.SemaphoreType.DMA,
        us1=pltpu.SemaphoreType.DMA,
        as0=pltpu.SemaphoreType.DMA,
        as1=pltpu.SemaphoreType.DMA,
    ),
)
def _layer_sc(
    xl_hbm, xr_hbm, src_hbm, dst_hbm, ea_hbm, deg_hbm, eas_hbm, w_hbm, a_hbm,
    u_out, as_out, *,
    u_sh, as_sh, xlb0, xlb1, xrb0, xrb1, srcb0, srcb1, dstb0, dstb1,
    eab0, eab1, sb0, sb1, wb, ab, azb, lad, idxb, d0b, d1b, e0b, e1b,
    mm0, mm1, gx0, gx1, gr0, gr1, us0, us1, as0, as1,
):
    c = lax.axis_index("c")
    s = lax.axis_index("s")
    wid = c * _NS + s
    lane = lax.iota(_i32, _L)
    zv = jnp.zeros((_L,), _f32)
    xlb = (xlb0, xlb1)
    xrb = (xrb0, xrb1)
    srcb = (srcb0, srcb1)
    dstb = (dstb0, dstb1)
    eab = (eab0, eab1)
    sb = (sb0, sb1)
    mm = (mm0, mm1)
    gx = (gx0, gx1)
    gr = (gr0, gr1)
    us = (us0, us1)
    asm = (as0, as1)

    _zero_rows(xlb0, _CH)
    _zero_flat(azb, _SLC)
    row0 = s * _SLC

    def zcp(r, _):
        pltpu.sync_copy(xlb0, u_sh.at[pl.ds(row0 + r * _CH, _CH), :])
        return 0

    lax.fori_loop(0, _SLC // _CH, zcp, 0)
    pltpu.sync_copy(azb, as_sh.at[pl.ds(row0, _SLC)])
    plsc.subcore_barrier()

    pltpu.sync_copy(w_hbm, wb)
    pltpu.sync_copy(a_hbm, ab)
    wv = [wb[pl.ds(k * _L, _L)] for k in range(_D // _L)]
    av = [ab[pl.ds(k * _L, _L)] for k in range(_D // _L)]
    for e in range(_L):
        lad[e, pl.ds(_L, _L)] = zv

    def _attention_groups(ngroups, xsrc, xdst, get_ea16, sbuf):
        def group(g, _):
            ea16 = get_ea16(g)
            alpha = zv
            for e in range(_L):
                eidx = g * _L + e
                eav = jnp.full((_L,), ea16[e], _f32)
                acc = None
                for k in range(_D // _L):
                    m = xsrc[eidx, pl.ds(k * _L, _L)] + xdst[eidx, pl.ds(k * _L, _L)]
                    m = m + eav * wv[k]
                    m = jnp.maximum(m, 0.2 * m)
                    t = m * av[k]
                    acc = t if acc is None else acc + t
                red = acc
                for sh in (8, 4, 2, 1):
                    lad[e, pl.ds(0, _L)] = red
                    red = red + lad[e, pl.ds(sh, _L)]
                alpha = jnp.where(lane == e, jnp.full((_L,), red[0], _f32), alpha)
            sv = jnp.exp(alpha)
            sbuf[pl.ds(g * _L, _L)] = sv
            for e in range(_L):
                eidx = g * _L + e
                se = jnp.full((_L,), sv[e], _f32)
                for k in range(_D // _L):
                    xsrc[eidx, pl.ds(k * _L, _L)] = xsrc[eidx, pl.ds(k * _L, _L)] * se
            return 0

        lax.fori_loop(0, ngroups, group, 0)

    ebase = wid * _EPT

    def step(ci, b):
        o = 1 - b

        @pl.when(ci >= 1)
        def _():
            pltpu.make_async_copy(xlb[o], u_sh.at[dstb[o]], us[o]).wait()
            pltpu.make_async_copy(sb[o], as_sh.at[dstb[o]], asm[o]).wait()

        @pl.when(ci + 1 < _NCHUNK)
        def _():
            nbase = ebase + (ci + 1) * _CH
            c1 = pltpu.async_copy(src_hbm.at[pl.ds(nbase, _CH)], srcb[o], mm[o])
            c2 = pltpu.async_copy(dst_hbm.at[pl.ds(nbase, _CH)], dstb[o], mm[o])
            c3 = pltpu.async_copy(ea_hbm.at[pl.ds(nbase, _CH)], eab[o], mm[o])
            c1.wait()
            c2.wait()
            c3.wait()
            pltpu.async_copy(xl_hbm.at[srcb[o]], xlb[o], gx[o])
            pltpu.async_copy(xr_hbm.at[dstb[o]], xrb[o], gr[o])

        pltpu.make_async_copy(xl_hbm.at[srcb[b]], xlb[b], gx[b]).wait()
        pltpu.make_async_copy(xr_hbm.at[dstb[b]], xrb[b], gr[b]).wait()
        _attention_groups(
            _CH // _L,
            xlb[b],
            xrb[b],
            lambda g: eab[b][pl.ds(g * _L, _L)],
            sb[b],
        )
        pltpu.async_copy(xlb[b], u_sh.at[dstb[b]], us[b], add=True)
        pltpu.async_copy(sb[b], as_sh.at[dstb[b]], asm[b], add=True)

    pltpu.sync_copy(src_hbm.at[pl.ds(ebase, _CH)], srcb0)
    pltpu.sync_copy(dst_hbm.at[pl.ds(ebase, _CH)], dstb0)
    pltpu.sync_copy(ea_hbm.at[pl.ds(ebase, _CH)], eab0)
    pltpu.async_copy(xl_hbm.at[srcb0], xlb0, gx0)
    pltpu.async_copy(xr_hbm.at[dstb0], xrb0, gr0)

    def pair(p, _):
        ci = 2 * p
        step(ci, 0)

        @pl.when(ci + 1 < _NCHUNK)
        def _():
            step(ci + 1, 1)

        return 0

    lax.fori_loop(0, (_NCHUNK + 1) // 2, pair, 0)
    pltpu.make_async_copy(xlb0, u_sh.at[dstb0], us0).wait()
    pltpu.make_async_copy(sb0, as_sh.at[dstb0], as0).wait()

    sc0 = wid * _NSC // _NW
    sc1 = (wid + 1) * _NSC // _NW

    def schunk(sc, _):
        nb = sc * _CH
        pltpu.sync_copy(xl_hbm.at[pl.ds(nb, _CH), :], xlb0)
        pltpu.sync_copy(xr_hbm.at[pl.ds(nb, _CH), :], xrb0)
        pltpu.sync_copy(deg_hbm.at[pl.ds(nb, _CH)], d0b)
        pltpu.sync_copy(deg_hbm.at[pl.ds(_NP + nb, _CH)], d1b)
        pltpu.sync_copy(eas_hbm.at[pl.ds(nb, _CH)], e0b)
        pltpu.sync_copy(eas_hbm.at[pl.ds(_NP + nb, _CH)], e1b)
        for i in range(_CH // _L):
            idxb[pl.ds(i * _L, _L)] = lane + (nb + i * _L)

        def lea16(g):
            dsl = pl.ds(g * _L, _L)
            deg = d0b[dsl] + d1b[dsl]
            return (e0b[dsl] + e1b[dsl]) / jnp.maximum(deg, 1.0)

        _attention_groups(_CH // _L, xlb0, xrb0, lea16, sb0)
        pltpu.sync_copy(xlb0, u_sh.at[idxb], add=True)
        pltpu.sync_copy(sb0, as_sh.at[idxb], add=True)
        return 0

    lax.fori_loop(sc0, sc1, schunk, 0)

    plsc.subcore_barrier()
    for r in range(_SLC // _D):
        pltpu.sync_copy(
            u_sh.at[pl.ds(row0 + r * _D, _D), :],
            u_out.at[c, pl.ds(row0 + r * _D, _D), :],
        )
    pltpu.sync_copy(as_sh.at[pl.ds(row0, _SLC)], as_out.at[pl.ds(c * _NP + row0, _SLC)])


_BLK = 640


def _proj_body(x_ref, wl_ref, bl_ref, wr_ref, br_ref, xl_ref, xr_ref):
    xv = x_ref[...]
    xl_ref[...] = jnp.dot(xv, wl_ref[...], preferred_element_type=_f32) + bl_ref[...]
    xr_ref[...] = jnp.dot(xv, wr_ref[...], preferred_element_type=_f32) + br_ref[...]


def _proj(x, wl, bl, wr, br):
    n = x.shape[0]
    blk = 1000
    assert n % blk == 0
    grid = n // blk
    wspec = pl.BlockSpec((_D, _D), lambda i: (0, 0))
    bspec = pl.BlockSpec((1, _D), lambda i: (0, 0))
    rspec = pl.BlockSpec((blk, _D), lambda i: (i, 0))
    return pl.pallas_call(
        _proj_body,
        grid=(grid,),
        in_specs=[rspec, wspec, bspec, wspec, bspec],
        out_specs=[rspec, rspec],
        out_shape=(
            jax.ShapeDtypeStruct((n, _D), _f32),
            jax.ShapeDtypeStruct((n, _D), _f32),
        ),
    )(x, wl, bl.reshape(1, _D), wr, br.reshape(1, _D))


def _fin_proj_body(u_ref, as_ref, b_ref, wl_ref, bl_ref, wr_ref, br_ref, xl_ref, xr_ref):
    usum = u_ref[0] + u_ref[1]
    ssum = as_ref[0] + as_ref[1]
    h = usum / jnp.maximum(ssum, 1e-35) + b_ref[...]
    g = 0.5 * h * (1.0 + lax.erf(h * (1.0 / math.sqrt(2.0))))
    xl_ref[...] = jnp.dot(g, wl_ref[...], preferred_element_type=_f32) + bl_ref[...]
    xr_ref[...] = jnp.dot(g, wr_ref[...], preferred_element_type=_f32) + br_ref[...]


def _fin_proj(u, asum, b, wl, bl, wr, br):
    grid = _NP // _BLK
    uspec = pl.BlockSpec((_NC, _BLK, _D), lambda i: (0, i, 0))
    aspec = pl.BlockSpec((_NC, _BLK, 1), lambda i: (0, i, 0))
    wspec = pl.BlockSpec((_D, _D), lambda i: (0, 0))
    bspec = pl.BlockSpec((1, _D), lambda i: (0, 0))
    rspec = pl.BlockSpec((_BLK, _D), lambda i: (i, 0))
    return pl.pallas_call(
        _fin_proj_body,
        grid=(grid,),
        in_specs=[uspec, aspec, bspec, wspec, bspec, wspec, bspec],
        out_specs=[rspec, rspec],
        out_shape=(
            jax.ShapeDtypeStruct((_NP, _D), _f32),
            jax.ShapeDtypeStruct((_NP, _D), _f32),
        ),
    )(u, asum.reshape(_NC, _NP, 1), b.reshape(1, _D), wl, bl.reshape(1, _D), wr, br.reshape(1, _D))


def _fin_body(u_ref, as_ref, b_ref, o_ref):
    usum = u_ref[0] + u_ref[1]
    ssum = as_ref[0] + as_ref[1]
    o_ref[...] = usum / jnp.maximum(ssum, 1e-35) + b_ref[...]


def _fin(u, asum, b):
    grid = _NP // _BLK
    uspec = pl.BlockSpec((_NC, _BLK, _D), lambda i: (0, i, 0))
    aspec = pl.BlockSpec((_NC, _BLK, 1), lambda i: (0, i, 0))
    bspec = pl.BlockSpec((1, _D), lambda i: (0, 0))
    rspec = pl.BlockSpec((_BLK, _D), lambda i: (i, 0))
    return pl.pallas_call(
        _fin_body,
        grid=(grid,),
        in_specs=[uspec, aspec, bspec],
        out_specs=rspec,
        out_shape=jax.ShapeDtypeStruct((_NP, _D), _f32),
    )(u, asum.reshape(_NC, _NP, 1), b.reshape(1, _D))


def kernel(x, e_i, e_a, Wl1, bl1, Wr1, br1, We1, att1, b1, Wl2, bl2, Wr2, br2, We2, att2, b2):
    src = e_i[0]
    dst = e_i[1]
    ea = e_a.reshape(-1)
    deg_p, eas_p = _prologue(dst, ea)
    xl1, xr1 = _proj(x, Wl1, bl1, Wr1, br1)
    xl1 = jnp.pad(xl1, ((0, _NP - _N), (0, 0)))
    xr1 = jnp.pad(xr1, ((0, _NP - _N), (0, 0)))
    u1, as1 = _layer_sc(
        xl1, xr1, src, dst, ea, deg_p, eas_p, We1.reshape(-1), att1.reshape(-1)
    )
    xl2, xr2 = _fin_proj(u1, as1.reshape(_NC, _NP), b1, Wl2, bl2, Wr2, br2)
    u2, as2 = _layer_sc(
        xl2, xr2, src, dst, ea, deg_p, eas_p, We2.reshape(-1), att2.reshape(-1)
    )
    out = _fin(u2, as2.reshape(_NC, _NP), b2)
    return out[:_N]
